# baseline (speedup 1.0000x reference)
import functools

import jax
import jax.numpy as jnp
from jax import lax
from jax.experimental import pallas as pl
from jax.experimental.pallas import tpu as pltpu
from jax.experimental.pallas import tpu_sc as plsc

N = 10000
E = 320000
D = 128
NPAD = 10240
E2 = 330240
EPW = E2 // 32
CD = 2064
CA = 80
NCH = EPW // CA
RB = 2000

_mesh = functools.partial(
    plsc.VectorSubcoreMesh, core_axis_name="c", subcore_axis_name="s")


@functools.partial(
    pl.kernel,
    out_type=jax.ShapeDtypeStruct((2 * NPAD,), jnp.float32),
    mesh=_mesh(),
    scratch_types=[
        pltpu.VMEM((CD,), jnp.int32),
        pltpu.VMEM((CD,), jnp.float32),
        pltpu.VMEM((640,), jnp.float32),
        pltpu.VMEM_SHARED((NPAD,), jnp.float32),
    ],
)
def _sc_degree(dst_hbm, out_hbm, idx_v, ones_v, zero_v, deg_sh):
    c = lax.axis_index("c")
    s = lax.axis_index("s")
    wid = c * 16 + s

    def fill_ones(i, carry):
        ones_v[pl.ds(i * 16, 16)] = jnp.full((16,), 1.0, jnp.float32)
        return carry

    lax.fori_loop(0, CD // 16, fill_ones, 0)

    def fill_zero(i, carry):
        zero_v[pl.ds(i * 16, 16)] = jnp.zeros((16,), jnp.float32)
        return carry

    lax.fori_loop(0, 640 // 16, fill_zero, 0)

    pltpu.sync_copy(zero_v, deg_sh.at[pl.ds(s * 640, 640)])
    plsc.subcore_barrier()

    def chunk(k, carry):
        base = wid * EPW + k * CD
        pltpu.sync_copy(dst_hbm.at[pl.ds(base, CD)], idx_v)
        pltpu.sync_copy(ones_v, deg_sh.at[idx_v], add=True)
        return carry

    lax.fori_loop(0, EPW // CD, chunk, 0)
    plsc.subcore_barrier()

    pltpu.sync_copy(deg_sh.at[pl.ds(s * 640, 640)],
                    out_hbm.at[pl.ds(c * NPAD + s * 640, 640)])


@functools.partial(
    pl.kernel,
    out_type=jax.ShapeDtypeStruct((2, NPAD, D), jnp.float32),
    mesh=_mesh(),
    scratch_types=[
        pltpu.VMEM((CA,), jnp.int32),
        pltpu.VMEM((CA,), jnp.int32),
        pltpu.VMEM((CA,), jnp.int32),
        pltpu.VMEM((CA,), jnp.int32),
        pltpu.VMEM((CA,), jnp.int32),
        pltpu.VMEM((CA,), jnp.int32),
        pltpu.VMEM((CA, D), jnp.float32),
        pltpu.VMEM((CA, D), jnp.float32),
        pltpu.VMEM((CA, D), jnp.float32),
        pltpu.SemaphoreType.DMA,
        pltpu.SemaphoreType.DMA,
        pltpu.SemaphoreType.DMA,
        pltpu.SemaphoreType.DMA,
        pltpu.SemaphoreType.DMA,
        pltpu.SemaphoreType.DMA,
        pltpu.SemaphoreType.DMA,
        pltpu.SemaphoreType.DMA,
        pltpu.SemaphoreType.DMA,
        pltpu.SemaphoreType.DMA,
        pltpu.SemaphoreType.DMA,
        pltpu.SemaphoreType.DMA,
        pltpu.VMEM_SHARED((NPAD, D), jnp.float32),
    ],
)
def _sc_aggregate(g_hbm, src_hbm, dst_hbm, out_hbm,
                  src0, src1, src2, dst0, dst1, dst2, rows0, rows1, rows2,
                  sg0, sg1, sg2, ss0, ss1, ss2, si0, si1, si2,
                  sd0, sd1, sd2, acc_sh):
    c = lax.axis_index("c")
    s = lax.axis_index("s")
    srcb = (src0, src1, src2)
    dstb = (dst0, dst1, dst2)
    rowsb = (rows0, rows1, rows2)
    sem_g = (sg0, sg1, sg2)
    sem_s = (ss0, ss1, ss2)
    sem_i = (si0, si1, si2)
    sem_d = (sd0, sd1, sd2)
    wid = c * 16 + s
    ebase = wid * EPW

    def fill_zero(i, carry):
        rows0[i // (D // 16), pl.ds((i % (D // 16)) * 16, 16)] = \
            jnp.zeros((16,), jnp.float32)
        return carry

    lax.fori_loop(0, CA * (D // 16), fill_zero, 0)

    row0 = s * 640
    for off in range(0, 640, CA):
        pltpu.sync_copy(rows0, acc_sh.at[pl.ds(row0 + off, CA)])
    plsc.subcore_barrier()

    def src_slice(k):
        return src_hbm.at[pl.ds(ebase + k * CA, CA)]

    def dst_slice(k):
        return dst_hbm.at[pl.ds(ebase + k * CA, CA)]

    pltpu.sync_copy(src_slice(0), src0)
    pltpu.async_copy(g_hbm.at[src0], rows0, sg0)
    pltpu.async_copy(dst_slice(0), dst0, sd0)
    pltpu.async_copy(src_slice(1), src1, si1)
    pltpu.async_copy(src_slice(2), src2, si2)

    def step(k, b):
        nb = (b + 1) % 3
        pltpu.make_async_copy(g_hbm.at[srcb[b]], rowsb[b], sem_g[b]).wait()
        pltpu.make_async_copy(dst_slice(k), dstb[b], sem_d[b]).wait()
        pltpu.async_copy(rowsb[b], acc_sh.at[dstb[b]], sem_s[b], add=True)

        @pl.when(k >= 2)
        def _():
            pltpu.make_async_copy(rowsb[nb], acc_sh.at[dstb[nb]],
                                  sem_s[nb]).wait()

        @pl.when(k + 1 < NCH)
        def _():
            pltpu.make_async_copy(src_slice(k + 1), srcb[nb], sem_i[nb]).wait()
            pltpu.async_copy(g_hbm.at[srcb[nb]], rowsb[nb], sem_g[nb])
            pltpu.async_copy(dst_slice(k + 1), dstb[nb], sem_d[nb])

        @pl.when(k + 3 < NCH)
        def _():
            pltpu.async_copy(src_slice(k + 3), srcb[b], sem_i[b])

    def triple(gt, carry):
        step(gt * 3, 0)
        step(gt * 3 + 1, 1)
        step(gt * 3 + 2, 2)
        return carry

    lax.fori_loop(0, NCH // 3, triple, 0)
    pltpu.make_async_copy(rowsb[(NCH - 2) % 3],
                          acc_sh.at[dstb[(NCH - 2) % 3]],
                          sem_s[(NCH - 2) % 3]).wait()
    pltpu.make_async_copy(rowsb[(NCH - 1) % 3],
                          acc_sh.at[dstb[(NCH - 1) % 3]],
                          sem_s[(NCH - 1) % 3]).wait()
    plsc.subcore_barrier()

    pltpu.sync_copy(acc_sh.at[pl.ds(row0, 640)],
                    out_hbm.at[c, pl.ds(row0, 640)])


def _norm_body(degp_ref, dis_ref):
    deg = degp_ref[0] + degp_ref[1]
    dis_ref[...] = lax.rsqrt(jnp.maximum(deg, 1.0))


def _tc_norm(deg_partials):
    degp = deg_partials.reshape(2, NPAD // D, D)
    return pl.pallas_call(
        _norm_body,
        out_shape=jax.ShapeDtypeStruct((NPAD // D, D), jnp.float32))(degp)


def _in_body(x_ref, w_ref, dis_ref, g_ref):
    h = jnp.dot(x_ref[...], w_ref[...], preferred_element_type=jnp.float32)
    g_ref[...] = h * dis_ref[...]


def _tc_in(x, w, dis):
    return pl.pallas_call(
        _in_body,
        grid=(N // RB,),
        in_specs=[
            pl.BlockSpec((RB, D), lambda i: (i, 0)),
            pl.BlockSpec((D, D), lambda i: (0, 0)),
            pl.BlockSpec((RB, 1), lambda i: (i, 0)),
        ],
        out_specs=pl.BlockSpec((RB, D), lambda i: (i, 0)),
        out_shape=jax.ShapeDtypeStruct((N, D), jnp.float32))(x, w, dis)


def _mid_body(a_ref, dis_ref, b_ref, w_ref, g_ref):
    xn = (a_ref[0] + a_ref[1]) * dis_ref[...] + b_ref[...]
    xn = jnp.maximum(xn, 0.0)
    h = jnp.dot(xn, w_ref[...], preferred_element_type=jnp.float32)
    g_ref[...] = h * dis_ref[...]


def _tc_mid(a, dis, b, w):
    return pl.pallas_call(
        _mid_body,
        grid=(N // RB,),
        in_specs=[
            pl.BlockSpec((2, RB, D), lambda i: (0, i, 0)),
            pl.BlockSpec((RB, 1), lambda i: (i, 0)),
            pl.BlockSpec((1, D), lambda i: (0, 0)),
            pl.BlockSpec((D, D), lambda i: (0, 0)),
        ],
        out_specs=pl.BlockSpec((RB, D), lambda i: (i, 0)),
        out_shape=jax.ShapeDtypeStruct((N, D), jnp.float32),
    )(a, dis, b.reshape(1, D), w)


def _out_body(a_ref, dis_ref, b_ref, o_ref):
    o_ref[...] = (a_ref[0] + a_ref[1]) * dis_ref[...] + b_ref[...]


def _tc_out(a, dis, b):
    return pl.pallas_call(
        _out_body,
        grid=(N // RB,),
        in_specs=[
            pl.BlockSpec((2, RB, D), lambda i: (0, i, 0)),
            pl.BlockSpec((RB, 1), lambda i: (i, 0)),
            pl.BlockSpec((1, D), lambda i: (0, 0)),
        ],
        out_specs=pl.BlockSpec((RB, D), lambda i: (i, 0)),
        out_shape=jax.ShapeDtypeStruct((N, D), jnp.float32),
    )(a, dis, b.reshape(1, D))


def kernel(x, edge_index, W1, b1, W2, b2, W3, b3):
    npadedge = E2 - E - N
    loop = jnp.arange(N, dtype=jnp.int32)
    src = jnp.concatenate([edge_index[0].astype(jnp.int32), loop,
                           jnp.arange(npadedge, dtype=jnp.int32)])
    dst = jnp.concatenate([edge_index[1].astype(jnp.int32), loop,
                           N + jnp.arange(npadedge, dtype=jnp.int32)])
    deg_partials = _sc_degree(dst)
    dis_pad = _tc_norm(deg_partials)
    dis = dis_pad.reshape(NPAD, 1)[:N]

    g1 = _tc_in(x, W1, dis)
    a = _sc_aggregate(g1, src, dst)
    g2 = _tc_mid(a, dis, b1, W2)
    a = _sc_aggregate(g2, src, dst)
    g3 = _tc_mid(a, dis, b2, W3)
    a = _sc_aggregate(g3, src, dst)
    return _tc_out(a, dis, b3)


# speedup vs baseline: 1.1488x; 1.1488x over previous
import functools

import jax
import jax.numpy as jnp
from jax import lax
from jax.experimental import pallas as pl
from jax.experimental.pallas import tpu as pltpu
from jax.experimental.pallas import tpu_sc as plsc

N = 10000
E = 320000
D = 128
NPAD = 10240
E2 = 330240
EPW = E2 // 32
CD = 2064
CA = 120
NCH = EPW // CA
RB = 2000

_mesh = functools.partial(
    plsc.VectorSubcoreMesh, core_axis_name="c", subcore_axis_name="s")


@functools.partial(
    pl.kernel,
    out_type=jax.ShapeDtypeStruct((2 * NPAD,), jnp.float32),
    mesh=_mesh(),
    scratch_types=[
        pltpu.VMEM((CD,), jnp.int32),
        pltpu.VMEM((CD,), jnp.float32),
        pltpu.VMEM((640,), jnp.float32),
        pltpu.VMEM_SHARED((NPAD,), jnp.float32),
    ],
)
def _sc_degree(dst_hbm, out_hbm, idx_v, ones_v, zero_v, deg_sh):
    c = lax.axis_index("c")
    s = lax.axis_index("s")
    wid = c * 16 + s

    def fill_ones(i, carry):
        ones_v[pl.ds(i * 16, 16)] = jnp.full((16,), 1.0, jnp.float32)
        return carry

    lax.fori_loop(0, CD // 16, fill_ones, 0)

    def fill_zero(i, carry):
        zero_v[pl.ds(i * 16, 16)] = jnp.zeros((16,), jnp.float32)
        return carry

    lax.fori_loop(0, 640 // 16, fill_zero, 0)

    pltpu.sync_copy(zero_v, deg_sh.at[pl.ds(s * 640, 640)])
    plsc.subcore_barrier()

    def chunk(k, carry):
        base = wid * EPW + k * CD
        pltpu.sync_copy(dst_hbm.at[pl.ds(base, CD)], idx_v)
        pltpu.sync_copy(ones_v, deg_sh.at[idx_v], add=True)
        return carry

    lax.fori_loop(0, EPW // CD, chunk, 0)
    plsc.subcore_barrier()

    pltpu.sync_copy(deg_sh.at[pl.ds(s * 640, 640)],
                    out_hbm.at[pl.ds(c * NPAD + s * 640, 640)])


@functools.partial(
    pl.kernel,
    out_type=jax.ShapeDtypeStruct((2, NPAD, D), jnp.float32),
    mesh=_mesh(),
    scratch_types=[
        pltpu.VMEM((CA,), jnp.int32),
        pltpu.VMEM((CA,), jnp.int32),
        pltpu.VMEM((CA,), jnp.int32),
        pltpu.VMEM((CA,), jnp.int32),
        pltpu.VMEM((CA,), jnp.int32),
        pltpu.VMEM((CA,), jnp.int32),
        pltpu.VMEM((CA, D), jnp.float32),
        pltpu.VMEM((CA, D), jnp.float32),
        pltpu.VMEM((CA, D), jnp.float32),
        pltpu.SemaphoreType.DMA,
        pltpu.SemaphoreType.DMA,
        pltpu.SemaphoreType.DMA,
        pltpu.SemaphoreType.DMA,
        pltpu.SemaphoreType.DMA,
        pltpu.SemaphoreType.DMA,
        pltpu.SemaphoreType.DMA,
        pltpu.SemaphoreType.DMA,
        pltpu.SemaphoreType.DMA,
        pltpu.SemaphoreType.DMA,
        pltpu.SemaphoreType.DMA,
        pltpu.SemaphoreType.DMA,
        pltpu.VMEM_SHARED((NPAD, D), jnp.float32),
    ],
)
def _sc_aggregate(g_hbm, src_hbm, dst_hbm, out_hbm,
                  src0, src1, src2, dst0, dst1, dst2, rows0, rows1, rows2,
                  sg0, sg1, sg2, ss0, ss1, ss2, si0, si1, si2,
                  sd0, sd1, sd2, acc_sh):
    c = lax.axis_index("c")
    s = lax.axis_index("s")
    srcb = (src0, src1, src2)
    dstb = (dst0, dst1, dst2)
    rowsb = (rows0, rows1, rows2)
    sem_g = (sg0, sg1, sg2)
    sem_s = (ss0, ss1, ss2)
    sem_i = (si0, si1, si2)
    sem_d = (sd0, sd1, sd2)
    wid = c * 16 + s
    ebase = wid * EPW

    def fill_zero(i, carry):
        rows0[i // (D // 16), pl.ds((i % (D // 16)) * 16, 16)] = \
            jnp.zeros((16,), jnp.float32)
        return carry

    lax.fori_loop(0, CA * (D // 16), fill_zero, 0)

    row0 = s * 640
    for off in range(0, 640 - CA + 1, CA):
        pltpu.sync_copy(rows0, acc_sh.at[pl.ds(row0 + off, CA)])
    rem = 640 % CA
    if rem:
        pltpu.sync_copy(rows0.at[pl.ds(0, rem)],
                        acc_sh.at[pl.ds(row0 + 640 - rem, rem)])
    plsc.subcore_barrier()

    def src_slice(k):
        return src_hbm.at[pl.ds(ebase + k * CA, CA)]

    def dst_slice(k):
        return dst_hbm.at[pl.ds(ebase + k * CA, CA)]

    pltpu.sync_copy(src_slice(0), src0)
    pltpu.async_copy(g_hbm.at[src0], rows0, sg0)
    pltpu.async_copy(dst_slice(0), dst0, sd0)
    pltpu.async_copy(src_slice(1), src1, si1)
    pltpu.async_copy(src_slice(2), src2, si2)

    def step(k, b):
        nb = (b + 1) % 3
        pltpu.make_async_copy(g_hbm.at[srcb[b]], rowsb[b], sem_g[b]).wait()
        pltpu.make_async_copy(dst_slice(k), dstb[b], sem_d[b]).wait()
        pltpu.async_copy(rowsb[b], acc_sh.at[dstb[b]], sem_s[b], add=True)

        @pl.when(k >= 2)
        def _():
            pltpu.make_async_copy(rowsb[nb], acc_sh.at[dstb[nb]],
                                  sem_s[nb]).wait()

        @pl.when(k + 1 < NCH)
        def _():
            pltpu.make_async_copy(src_slice(k + 1), srcb[nb], sem_i[nb]).wait()
            pltpu.async_copy(g_hbm.at[srcb[nb]], rowsb[nb], sem_g[nb])
            pltpu.async_copy(dst_slice(k + 1), dstb[nb], sem_d[nb])

        @pl.when(k + 3 < NCH)
        def _():
            pltpu.async_copy(src_slice(k + 3), srcb[b], sem_i[b])

    def triple(gt, carry):
        step(gt * 3, 0)
        step(gt * 3 + 1, 1)
        step(gt * 3 + 2, 2)
        return carry

    lax.fori_loop(0, NCH // 3, triple, 0)
    for k in range(NCH - NCH % 3, NCH):
        step(k, k % 3)
    pltpu.make_async_copy(rowsb[(NCH - 2) % 3],
                          acc_sh.at[dstb[(NCH - 2) % 3]],
                          sem_s[(NCH - 2) % 3]).wait()
    pltpu.make_async_copy(rowsb[(NCH - 1) % 3],
                          acc_sh.at[dstb[(NCH - 1) % 3]],
                          sem_s[(NCH - 1) % 3]).wait()
    plsc.subcore_barrier()

    pltpu.sync_copy(acc_sh.at[pl.ds(row0, 640)],
                    out_hbm.at[c, pl.ds(row0, 640)])


def _norm_body(degp_ref, dis_ref):
    deg = degp_ref[0] + degp_ref[1]
    dis_ref[...] = lax.rsqrt(jnp.maximum(deg, 1.0))


def _tc_norm(deg_partials):
    degp = deg_partials.reshape(2, NPAD // D, D)
    return pl.pallas_call(
        _norm_body,
        out_shape=jax.ShapeDtypeStruct((NPAD // D, D), jnp.float32))(degp)


def _in_body(x_ref, w_ref, dis_ref, g_ref):
    h = jnp.dot(x_ref[...], w_ref[...], preferred_element_type=jnp.float32)
    g_ref[...] = h * dis_ref[...]


def _tc_in(x, w, dis):
    return pl.pallas_call(
        _in_body,
        grid=(N // RB,),
        in_specs=[
            pl.BlockSpec((RB, D), lambda i: (i, 0)),
            pl.BlockSpec((D, D), lambda i: (0, 0)),
            pl.BlockSpec((RB, 1), lambda i: (i, 0)),
        ],
        out_specs=pl.BlockSpec((RB, D), lambda i: (i, 0)),
        out_shape=jax.ShapeDtypeStruct((N, D), jnp.float32))(x, w, dis)


def _mid_body(a_ref, dis_ref, b_ref, w_ref, g_ref):
    xn = (a_ref[0] + a_ref[1]) * dis_ref[...] + b_ref[...]
    xn = jnp.maximum(xn, 0.0)
    h = jnp.dot(xn, w_ref[...], preferred_element_type=jnp.float32)
    g_ref[...] = h * dis_ref[...]


def _tc_mid(a, dis, b, w):
    return pl.pallas_call(
        _mid_body,
        grid=(N // RB,),
        in_specs=[
            pl.BlockSpec((2, RB, D), lambda i: (0, i, 0)),
            pl.BlockSpec((RB, 1), lambda i: (i, 0)),
            pl.BlockSpec((1, D), lambda i: (0, 0)),
            pl.BlockSpec((D, D), lambda i: (0, 0)),
        ],
        out_specs=pl.BlockSpec((RB, D), lambda i: (i, 0)),
        out_shape=jax.ShapeDtypeStruct((N, D), jnp.float32),
    )(a, dis, b.reshape(1, D), w)


def _out_body(a_ref, dis_ref, b_ref, o_ref):
    o_ref[...] = (a_ref[0] + a_ref[1]) * dis_ref[...] + b_ref[...]


def _tc_out(a, dis, b):
    return pl.pallas_call(
        _out_body,
        grid=(N // RB,),
        in_specs=[
            pl.BlockSpec((2, RB, D), lambda i: (0, i, 0)),
            pl.BlockSpec((RB, 1), lambda i: (i, 0)),
            pl.BlockSpec((1, D), lambda i: (0, 0)),
        ],
        out_specs=pl.BlockSpec((RB, D), lambda i: (i, 0)),
        out_shape=jax.ShapeDtypeStruct((N, D), jnp.float32),
    )(a, dis, b.reshape(1, D))


def kernel(x, edge_index, W1, b1, W2, b2, W3, b3):
    npadedge = E2 - E - N
    loop = jnp.arange(N, dtype=jnp.int32)
    src = jnp.concatenate([edge_index[0].astype(jnp.int32), loop,
                           jnp.arange(npadedge, dtype=jnp.int32)])
    dst = jnp.concatenate([edge_index[1].astype(jnp.int32), loop,
                           N + jnp.arange(npadedge, dtype=jnp.int32)])
    deg_partials = _sc_degree(dst)
    dis_pad = _tc_norm(deg_partials)
    dis = dis_pad.reshape(NPAD, 1)[:N]

    g1 = _tc_in(x, W1, dis)
    a = _sc_aggregate(g1, src, dst)
    g2 = _tc_mid(a, dis, b1, W2)
    a = _sc_aggregate(g2, src, dst)
    g3 = _tc_mid(a, dis, b2, W3)
    a = _sc_aggregate(g3, src, dst)
    return _tc_out(a, dis, b3)


# speedup vs baseline: 1.2223x; 1.0639x over previous
import functools

import jax
import jax.numpy as jnp
from jax import lax
from jax.experimental import pallas as pl
from jax.experimental.pallas import tpu as pltpu
from jax.experimental.pallas import tpu_sc as plsc

N = 10000
E = 320000
D = 128
STRIPE = 632
NPAD = 10112
NPD = 10240
E2 = 330752
EPW = E2 // 32
CD = 2584
CA = 152
NCH = EPW // CA
RB = 2000

_mesh = functools.partial(
    plsc.VectorSubcoreMesh, core_axis_name="c", subcore_axis_name="s")


@functools.partial(
    pl.kernel,
    out_type=jax.ShapeDtypeStruct((2 * NPD,), jnp.float32),
    mesh=_mesh(),
    scratch_types=[
        pltpu.VMEM((CD,), jnp.int32),
        pltpu.VMEM((CD,), jnp.float32),
        pltpu.VMEM((640,), jnp.float32),
        pltpu.VMEM_SHARED((NPD,), jnp.float32),
    ],
)
def _sc_degree(dst_hbm, out_hbm, idx_v, ones_v, zero_v, deg_sh):
    c = lax.axis_index("c")
    s = lax.axis_index("s")
    wid = c * 16 + s

    def fill_ones(i, carry):
        ones_v[pl.ds(i * 16, 16)] = jnp.full((16,), 1.0, jnp.float32)
        return carry

    lax.fori_loop(0, CD // 16, fill_ones, 0)

    def fill_zero(i, carry):
        zero_v[pl.ds(i * 16, 16)] = jnp.zeros((16,), jnp.float32)
        return carry

    lax.fori_loop(0, 640 // 16, fill_zero, 0)

    pltpu.sync_copy(zero_v, deg_sh.at[pl.ds(s * 640, 640)])
    plsc.subcore_barrier()

    def chunk(k, carry):
        base = wid * EPW + k * CD
        pltpu.sync_copy(dst_hbm.at[pl.ds(base, CD)], idx_v)
        pltpu.sync_copy(ones_v, deg_sh.at[idx_v], add=True)
        return carry

    lax.fori_loop(0, EPW // CD, chunk, 0)
    plsc.subcore_barrier()

    pltpu.sync_copy(deg_sh.at[pl.ds(s * 640, 640)],
                    out_hbm.at[pl.ds(c * NPD + s * 640, 640)])


@functools.partial(
    pl.kernel,
    out_type=jax.ShapeDtypeStruct((2, NPAD, D), jnp.float32),
    mesh=_mesh(),
    scratch_types=[
        pltpu.VMEM((CA,), jnp.int32),
        pltpu.VMEM((CA,), jnp.int32),
        pltpu.VMEM((CA,), jnp.int32),
        pltpu.VMEM((CA,), jnp.int32),
        pltpu.VMEM((CA, D), jnp.float32),
        pltpu.VMEM((CA, D), jnp.float32),
        pltpu.SemaphoreType.DMA,
        pltpu.SemaphoreType.DMA,
        pltpu.SemaphoreType.DMA,
        pltpu.SemaphoreType.DMA,
        pltpu.SemaphoreType.DMA,
        pltpu.SemaphoreType.DMA,
        pltpu.SemaphoreType.DMA,
        pltpu.SemaphoreType.DMA,
        pltpu.VMEM_SHARED((NPAD, D), jnp.float32),
    ],
)
def _sc_aggregate(g_hbm, src_hbm, dst_hbm, out_hbm,
                  src0, src1, dst0, dst1, rows0, rows1,
                  sg0, sg1, ss0, ss1, si0, si1, sd0, sd1, acc_sh):
    c = lax.axis_index("c")
    s = lax.axis_index("s")
    srcb = (src0, src1)
    dstb = (dst0, dst1)
    rowsb = (rows0, rows1)
    sem_g = (sg0, sg1)
    sem_s = (ss0, ss1)
    sem_i = (si0, si1)
    sem_d = (sd0, sd1)
    wid = c * 16 + s
    ebase = wid * EPW

    def fill_zero(i, carry):
        rows0[i // (D // 16), pl.ds((i % (D // 16)) * 16, 16)] = \
            jnp.zeros((16,), jnp.float32)
        return carry

    lax.fori_loop(0, CA * (D // 16), fill_zero, 0)

    row0 = s * STRIPE
    for off in range(0, STRIPE - CA + 1, CA):
        pltpu.sync_copy(rows0, acc_sh.at[pl.ds(row0 + off, CA)])
    rem = STRIPE % CA
    if rem:
        pltpu.sync_copy(rows0.at[pl.ds(0, rem)],
                        acc_sh.at[pl.ds(row0 + STRIPE - rem, rem)])
    plsc.subcore_barrier()

    def src_slice(k):
        return src_hbm.at[pl.ds(ebase + k * CA, CA)]

    def dst_slice(k):
        return dst_hbm.at[pl.ds(ebase + k * CA, CA)]

    pltpu.sync_copy(src_slice(0), src0)
    pltpu.async_copy(g_hbm.at[src0], rows0, sg0)
    pltpu.async_copy(dst_slice(0), dst0, sd0)
    pltpu.async_copy(src_slice(1), src1, si1)

    def step(k, b):
        nb = 1 - b
        pltpu.make_async_copy(g_hbm.at[srcb[b]], rowsb[b], sem_g[b]).wait()
        pltpu.make_async_copy(dst_slice(k), dstb[b], sem_d[b]).wait()
        pltpu.async_copy(rowsb[b], acc_sh.at[dstb[b]], sem_s[b], add=True)

        @pl.when(k >= 1)
        def _():
            pltpu.make_async_copy(rowsb[nb], acc_sh.at[dstb[nb]],
                                  sem_s[nb]).wait()

        @pl.when(k + 1 < NCH)
        def _():
            pltpu.make_async_copy(src_slice(k + 1), srcb[nb], sem_i[nb]).wait()
            pltpu.async_copy(g_hbm.at[srcb[nb]], rowsb[nb], sem_g[nb])
            pltpu.async_copy(dst_slice(k + 1), dstb[nb], sem_d[nb])

        @pl.when(k + 2 < NCH)
        def _():
            pltpu.async_copy(src_slice(k + 2), srcb[b], sem_i[b])

    def pair(gp, carry):
        step(gp * 2, 0)
        step(gp * 2 + 1, 1)
        return carry

    lax.fori_loop(0, NCH // 2, pair, 0)
    pltpu.make_async_copy(rowsb[(NCH - 1) % 2],
                          acc_sh.at[dstb[(NCH - 1) % 2]],
                          sem_s[(NCH - 1) % 2]).wait()
    plsc.subcore_barrier()

    pltpu.sync_copy(acc_sh.at[pl.ds(row0, STRIPE)],
                    out_hbm.at[c, pl.ds(row0, STRIPE)])


def _norm_body(degp_ref, dis_ref):
    deg = degp_ref[0] + degp_ref[1]
    dis_ref[...] = lax.rsqrt(jnp.maximum(deg, 1.0))


def _tc_norm(deg_partials):
    degp = deg_partials.reshape(2, NPD // D, D)
    return pl.pallas_call(
        _norm_body,
        out_shape=jax.ShapeDtypeStruct((NPD // D, D), jnp.float32))(degp)


def _in_body(x_ref, w_ref, dis_ref, g_ref):
    h = jnp.dot(x_ref[...], w_ref[...], preferred_element_type=jnp.float32)
    g_ref[...] = h * dis_ref[...]


def _tc_in(x, w, dis):
    return pl.pallas_call(
        _in_body,
        grid=(N // RB,),
        in_specs=[
            pl.BlockSpec((RB, D), lambda i: (i, 0)),
            pl.BlockSpec((D, D), lambda i: (0, 0)),
            pl.BlockSpec((RB, 1), lambda i: (i, 0)),
        ],
        out_specs=pl.BlockSpec((RB, D), lambda i: (i, 0)),
        out_shape=jax.ShapeDtypeStruct((N, D), jnp.float32))(x, w, dis)


def _mid_body(a_ref, dis_ref, b_ref, w_ref, g_ref):
    xn = (a_ref[0] + a_ref[1]) * dis_ref[...] + b_ref[...]
    xn = jnp.maximum(xn, 0.0)
    h = jnp.dot(xn, w_ref[...], preferred_element_type=jnp.float32)
    g_ref[...] = h * dis_ref[...]


def _tc_mid(a, dis, b, w):
    return pl.pallas_call(
        _mid_body,
        grid=(N // RB,),
        in_specs=[
            pl.BlockSpec((2, RB, D), lambda i: (0, i, 0)),
            pl.BlockSpec((RB, 1), lambda i: (i, 0)),
            pl.BlockSpec((1, D), lambda i: (0, 0)),
            pl.BlockSpec((D, D), lambda i: (0, 0)),
        ],
        out_specs=pl.BlockSpec((RB, D), lambda i: (i, 0)),
        out_shape=jax.ShapeDtypeStruct((N, D), jnp.float32),
    )(a, dis, b.reshape(1, D), w)


def _out_body(a_ref, dis_ref, b_ref, o_ref):
    o_ref[...] = (a_ref[0] + a_ref[1]) * dis_ref[...] + b_ref[...]


def _tc_out(a, dis, b):
    return pl.pallas_call(
        _out_body,
        grid=(N // RB,),
        in_specs=[
            pl.BlockSpec((2, RB, D), lambda i: (0, i, 0)),
            pl.BlockSpec((RB, 1), lambda i: (i, 0)),
            pl.BlockSpec((1, D), lambda i: (0, 0)),
        ],
        out_specs=pl.BlockSpec((RB, D), lambda i: (i, 0)),
        out_shape=jax.ShapeDtypeStruct((N, D), jnp.float32),
    )(a, dis, b.reshape(1, D))


def kernel(x, edge_index, W1, b1, W2, b2, W3, b3):
    npadedge = E2 - E - N
    ndump = NPAD - N
    loop = jnp.arange(N, dtype=jnp.int32)
    src = jnp.concatenate([edge_index[0].astype(jnp.int32), loop,
                           jnp.arange(npadedge, dtype=jnp.int32)])
    dst = jnp.concatenate([edge_index[1].astype(jnp.int32), loop,
                           N + jnp.arange(npadedge, dtype=jnp.int32) % ndump])
    deg_partials = _sc_degree(dst)
    dis_pad = _tc_norm(deg_partials)
    dis = dis_pad.reshape(NPD, 1)[:N]

    g1 = _tc_in(x, W1, dis)
    a = _sc_aggregate(g1, src, dst)
    g2 = _tc_mid(a, dis, b1, W2)
    a = _sc_aggregate(g2, src, dst)
    g3 = _tc_mid(a, dis, b2, W3)
    a = _sc_aggregate(g3, src, dst)
    return _tc_out(a, dis, b3)
